# baseline (speedup 1.0000x reference)
import functools

import jax
import jax.numpy as jnp
from jax import lax
from jax.experimental import pallas as pl
from jax.experimental.pallas import tpu as pltpu
from jax.experimental.pallas import tpu_sc as plsc

_B, _S, _V, _D = 4096, 77, 49408, 1024
_N = _B * _S
_LANES = 16
_K = 32
_NBUF = 3
_PF = 2


def _make_sc_kernel():
    info = plsc.get_sparse_core_info()
    num_cores, num_subcores = info.num_cores, info.num_subcores
    nw = num_cores * num_subcores
    seq_per_w = _B // nw
    b_per_w = _N // nw
    jblocks = seq_per_w // _K
    n_chunks = _S * jblocks
    n_main = (n_chunks // _NBUF) * _NBUF
    mesh = plsc.VectorSubcoreMesh(core_axis_name="c", subcore_axis_name="s")

    @functools.partial(
        pl.kernel,
        out_type=jax.ShapeDtypeStruct((_N, _D), jnp.float32),
        mesh=mesh,
        scratch_types=[
            pltpu.VMEM((b_per_w,), jnp.int32),
            pltpu.VMEM((_D,), jnp.float32),
            [pltpu.VMEM((_K, _D), jnp.float32) for _ in range(_NBUF)],
            [pltpu.SemaphoreType.DMA for _ in range(_NBUF)],
            [pltpu.SemaphoreType.DMA for _ in range(_NBUF)],
        ],
    )
    def sc_kernel(idx_hbm, table_hbm, pos_hbm, out_hbm,
                  idx_v, pos_v, bufs, gsems, ssems):
        wid = lax.axis_index("s") * num_cores + lax.axis_index("c")
        out_base = wid * b_per_w
        pltpu.sync_copy(idx_hbm.at[wid], idx_v)

        def chunk_s(c):
            return c // jblocks

        def chunk_j0(c):
            return (c % jblocks) * _K

        def out_idx(c, half):
            lane = lax.iota(jnp.int32, _LANES)
            return ((lane + (chunk_j0(c) + half * _LANES)) * _S
                    + (out_base + chunk_s(c)))

        def start_gather(c, b):
            pltpu.async_copy(
                table_hbm.at[idx_v.at[pl.ds(c * _K, _K)]], bufs[b], gsems[b])

        def wait_gather(c, b):
            pltpu.make_async_copy(
                table_hbm.at[idx_v.at[pl.ds(c * _K, _K)]], bufs[b],
                gsems[b]).wait()

        def start_store(c, b):
            for h in range(2):
                pltpu.async_copy(bufs[b].at[pl.ds(h * _LANES, _LANES)],
                                 out_hbm.at[out_idx(c, h)], ssems[b])

        def wait_store(c, b):
            for h in range(2):
                pltpu.make_async_copy(bufs[b].at[pl.ds(h * _LANES, _LANES)],
                                      out_hbm.at[out_idx(c, h)],
                                      ssems[b]).wait()

        def add_pos(b):
            rows_v = bufs[b]

            def body(i, _):
                sl = pl.ds(i * _LANES, _LANES)
                p = pos_v[sl]
                for r in range(_K):
                    rows_v[r, sl] = rows_v[r, sl] + p
                return _

            lax.fori_loop(0, _D // _LANES, body, None)

        pltpu.sync_copy(pos_hbm.at[0], pos_v)
        for c in range(_PF):
            start_gather(c, c)

        def process(c, b):
            @pl.when(c % jblocks == 0)
            def _():
                pltpu.sync_copy(pos_hbm.at[chunk_s(c)], pos_v)

            wait_gather(c, b)
            add_pos(b)
            start_store(c, b)
            bn = (b + _PF) % _NBUF

            @pl.when(c + _PF < n_chunks)
            def _():
                @pl.when(c >= _NBUF - _PF)
                def _():
                    wait_store(c + _PF - _NBUF, bn)

                start_gather(c + _PF, bn)

        def body(g, _):
            for b in range(_NBUF):
                process(g * _NBUF + b, b)
            return _

        lax.fori_loop(0, n_main // _NBUF, body, None)
        for c in range(n_main, n_chunks):
            process(jnp.int32(c), c % _NBUF)
        for c in range(n_chunks - _NBUF, n_chunks):
            wait_store(c, c % _NBUF)

    return sc_kernel


_sc_kernel = _make_sc_kernel()


@jax.jit
def kernel(input_tokens, token_table, pos_table):
    info = plsc.get_sparse_core_info()
    nw = info.num_cores * info.num_subcores
    idx = (input_tokens.astype(jnp.int32)
           .reshape(nw, _B // nw, _S)
           .transpose(0, 2, 1)
           .reshape(nw, -1))
    out = _sc_kernel(idx, token_table, pos_table.astype(jnp.float32))
    return out.reshape(_B, _S, _D)
